# Initial kernel scaffold; baseline (speedup 1.0000x reference)
import jax
import jax.numpy as jnp
from jax.experimental import pallas as pl


def kernel(x, table):
    raise NotImplementedError("write your pallas kernel here")


# speedup vs baseline: 4.7128x; 4.7128x over previous
import functools

import jax
import jax.numpy as jnp
from jax import lax
from jax.experimental import pallas as pl
from jax.experimental.pallas import tpu as pltpu
from jax.experimental.pallas import tpu_sc as plsc


@functools.lru_cache(maxsize=None)
def _build_gather(B, D):
    info = plsc.get_sparse_core_info()
    NC, NS = info.num_cores, info.num_subcores
    NW = NC * NS
    assert B % NW == 0
    b_per_w = B // NW
    C = 800
    assert b_per_w % C == 0 and C % 8 == 0
    n_chunks = b_per_w // C

    mesh = plsc.VectorSubcoreMesh(core_axis_name="c", subcore_axis_name="s")

    @functools.partial(
        pl.kernel,
        mesh=mesh,
        out_type=jax.ShapeDtypeStruct((B, D), jnp.float32),
        scratch_types=[
            pltpu.VMEM((C,), jnp.int32),
            pltpu.VMEM((C, D), jnp.float32),
            pltpu.SemaphoreType.DMA,
        ],
        compiler_params=pltpu.CompilerParams(use_tc_tiling_on_sc=False),
    )
    def gather_kernel(idx_hbm, table_hbm, out_hbm, idx_v, rows_v, sem):
        wid = lax.axis_index("s") * NC + lax.axis_index("c")
        base = wid * b_per_w

        def body(i, carry):
            start = base + i * C
            pltpu.sync_copy(idx_hbm.at[pl.ds(start, C)], idx_v)
            pltpu.async_copy(table_hbm.at[idx_v], rows_v, sem).wait()
            pltpu.sync_copy(rows_v, out_hbm.at[pl.ds(start, C)])
            return carry

        lax.fori_loop(0, n_chunks, body, 0)

    return gather_kernel


def kernel(x, table):
    bsz, hist = x.shape
    d = table.shape[1]
    idx = x.reshape(-1).astype(jnp.int32)
    out = _build_gather(bsz * hist, d)(idx, table)
    return out.reshape(bsz, hist, d)


# speedup vs baseline: 4.7391x; 1.0056x over previous
import functools

import jax
import jax.numpy as jnp
from jax import lax
from jax.experimental import pallas as pl
from jax.experimental.pallas import tpu as pltpu
from jax.experimental.pallas import tpu_sc as plsc


@functools.lru_cache(maxsize=None)
def _build_gather(B, D):
    info = plsc.get_sparse_core_info()
    NC, NS = info.num_cores, info.num_subcores
    NW = NC * NS
    assert B % NW == 0
    b_per_w = B // NW
    C = 800
    assert b_per_w % (2 * C) == 0 and C % 8 == 0
    n_groups = b_per_w // (2 * C)

    mesh = plsc.VectorSubcoreMesh(core_axis_name="c", subcore_axis_name="s")

    @functools.partial(
        pl.kernel,
        mesh=mesh,
        out_type=jax.ShapeDtypeStruct((B, D), jnp.float32),
        scratch_types=[
            pltpu.VMEM((C,), jnp.int32),
            pltpu.VMEM((C,), jnp.int32),
            pltpu.VMEM((C, D), jnp.float32),
            pltpu.VMEM((C, D), jnp.float32),
            pltpu.SemaphoreType.DMA,
            pltpu.SemaphoreType.DMA,
            pltpu.SemaphoreType.DMA,
            pltpu.SemaphoreType.DMA,
        ],
        compiler_params=pltpu.CompilerParams(use_tc_tiling_on_sc=False),
    )
    def gather_kernel(idx_hbm, table_hbm, out_hbm, idx0, idx1, rows0, rows1,
                      sg0, sg1, sw0, sw1):
        wid = lax.axis_index("s") * NC + lax.axis_index("c")
        base = wid * b_per_w

        pltpu.sync_copy(idx_hbm.at[pl.ds(base, C)], idx0)
        g0 = pltpu.async_copy(table_hbm.at[idx0], rows0, sg0)

        def body(g, carry):
            i0 = base + (2 * g) * C
            i1 = i0 + C
            pltpu.sync_copy(idx_hbm.at[pl.ds(i1, C)], idx1)
            pltpu.make_async_copy(table_hbm.at[idx0], rows0, sg0).wait()

            @pl.when(g > 0)
            def _():
                pltpu.make_async_copy(rows1, out_hbm.at[pl.ds(i1 - 2 * C, C)],
                                      sw1).wait()

            pltpu.async_copy(rows0, out_hbm.at[pl.ds(i0, C)], sw0)
            pltpu.async_copy(table_hbm.at[idx1], rows1, sg1)

            @pl.when(g < n_groups - 1)
            def _():
                pltpu.sync_copy(idx_hbm.at[pl.ds(i0 + 2 * C, C)], idx0)

            pltpu.make_async_copy(table_hbm.at[idx1], rows1, sg1).wait()
            pltpu.make_async_copy(rows0, out_hbm.at[pl.ds(i0, C)], sw0).wait()
            pltpu.async_copy(rows1, out_hbm.at[pl.ds(i1, C)], sw1)

            @pl.when(g < n_groups - 1)
            def _():
                pltpu.async_copy(table_hbm.at[idx0], rows0, sg0)

            return carry

        lax.fori_loop(0, n_groups, body, 0)
        last = base + b_per_w - C
        pltpu.make_async_copy(rows1, out_hbm.at[pl.ds(last, C)], sw1).wait()

    return gather_kernel


def kernel(x, table):
    bsz, hist = x.shape
    d = table.shape[1]
    idx = x.reshape(-1).astype(jnp.int32)
    out = _build_gather(bsz * hist, d)(idx, table)
    return out.reshape(bsz, hist, d)


# speedup vs baseline: 4.7420x; 1.0006x over previous
import functools

import jax
import jax.numpy as jnp
from jax import lax
from jax.experimental import pallas as pl
from jax.experimental.pallas import tpu as pltpu
from jax.experimental.pallas import tpu_sc as plsc


@functools.lru_cache(maxsize=None)
def _build_gather(BSZ, HIST, D):
    info = plsc.get_sparse_core_info()
    NC, NS = info.num_cores, info.num_subcores
    NW = NC * NS
    assert BSZ % NW == 0
    rows_per_w = BSZ // NW
    R = 4
    C = R * HIST
    assert rows_per_w % (2 * R) == 0 and C % 8 == 0
    n_groups = rows_per_w // (2 * R)

    mesh = plsc.VectorSubcoreMesh(core_axis_name="c", subcore_axis_name="s")

    @functools.partial(
        pl.kernel,
        mesh=mesh,
        out_type=jax.ShapeDtypeStruct((BSZ, HIST, D), jnp.float32),
        scratch_types=[
            pltpu.VMEM((C,), jnp.int32),
            pltpu.VMEM((C,), jnp.int32),
            pltpu.VMEM((C, D), jnp.float32),
            pltpu.VMEM((C, D), jnp.float32),
            pltpu.SemaphoreType.DMA,
            pltpu.SemaphoreType.DMA,
            pltpu.SemaphoreType.DMA,
            pltpu.SemaphoreType.DMA,
        ],
        compiler_params=pltpu.CompilerParams(use_tc_tiling_on_sc=False),
    )
    def gather_kernel(idx_hbm, table_hbm, out_hbm, idx0, idx1, rows0, rows1,
                      sg0, sg1, sw0, sw1):
        wid = lax.axis_index("s") * NC + lax.axis_index("c")
        base = wid * rows_per_w

        def start_write(rows, r, sem):
            for k in range(R):
                pltpu.async_copy(rows.at[pl.ds(k * HIST, HIST)],
                                 out_hbm.at[r + k], sem)

        def wait_write(rows, r, sem):
            for k in range(R):
                pltpu.make_async_copy(rows.at[pl.ds(k * HIST, HIST)],
                                      out_hbm.at[r + k], sem).wait()

        pltpu.sync_copy(idx_hbm.at[pl.ds(base * HIST, C)], idx0)
        pltpu.async_copy(table_hbm.at[idx0], rows0, sg0)

        def body(g, carry):
            r0 = base + (2 * g) * R
            r1 = r0 + R
            pltpu.sync_copy(idx_hbm.at[pl.ds(r1 * HIST, C)], idx1)
            pltpu.make_async_copy(table_hbm.at[idx0], rows0, sg0).wait()

            @pl.when(g > 0)
            def _():
                wait_write(rows1, r1 - 2 * R, sw1)

            start_write(rows0, r0, sw0)
            pltpu.async_copy(table_hbm.at[idx1], rows1, sg1)

            @pl.when(g < n_groups - 1)
            def _():
                pltpu.sync_copy(idx_hbm.at[pl.ds((r0 + 2 * R) * HIST, C)],
                                idx0)

            pltpu.make_async_copy(table_hbm.at[idx1], rows1, sg1).wait()
            wait_write(rows0, r0, sw0)
            start_write(rows1, r1, sw1)

            @pl.when(g < n_groups - 1)
            def _():
                pltpu.async_copy(table_hbm.at[idx0], rows0, sg0)

            return carry

        lax.fori_loop(0, n_groups, body, 0)
        wait_write(rows1, base + rows_per_w - R, sw1)

    return gather_kernel


def kernel(x, table):
    bsz, hist = x.shape
    d = table.shape[1]
    idx = x.reshape(-1).astype(jnp.int32)
    return _build_gather(bsz, hist, d)(idx, table)


# speedup vs baseline: 9.6605x; 2.0372x over previous
import functools

import jax
import jax.numpy as jnp
from jax import lax
from jax.experimental import pallas as pl
from jax.experimental.pallas import tpu as pltpu
from jax.experimental.pallas import tpu_sc as plsc

_L = 128
_RING = 4
_BLK = 80


@functools.lru_cache(maxsize=None)
def _build_gather(B, D):
    info = plsc.get_sparse_core_info()
    NC, NS = info.num_cores, info.num_subcores
    NW = NC * NS
    FR = B // _L
    assert B % _L == 0 and FR % NW == 0
    fr_per_w = FR // NW
    assert fr_per_w % _BLK == 0 and _BLK % _RING == 0
    n_blocks = fr_per_w // _BLK
    n_groups = _BLK // _RING

    mesh = plsc.VectorSubcoreMesh(core_axis_name="c", subcore_axis_name="s")

    @functools.partial(
        pl.kernel,
        mesh=mesh,
        out_type=jax.ShapeDtypeStruct((B, D), jnp.float32),
        scratch_types=[
            pltpu.VMEM((_BLK, _L), jnp.int32),
            [pltpu.VMEM((_L, D), jnp.float32) for _ in range(_RING)],
            pltpu.VMEM_SHARED((2048, D), jnp.float32),
            [pltpu.SemaphoreType.DMA for _ in range(_RING)],
            [pltpu.SemaphoreType.DMA for _ in range(_RING)],
        ],
        compiler_params=pltpu.CompilerParams(use_tc_tiling_on_sc=True),
    )
    def gather_kernel(idx_hbm, table_hbm, out_hbm, idx_v, rows, shared_tab,
                      sg, sw):
        wid = lax.axis_index("s") * NC + lax.axis_index("c")
        base_fr = wid * fr_per_w

        @pl.when(lax.axis_index("s") == 0)
        def _():
            pltpu.sync_copy(table_hbm, shared_tab)

        plsc.subcore_barrier()

        def block(bi, carry):
            fr0 = base_fr + bi * _BLK
            pltpu.sync_copy(idx_hbm.at[pl.ds(fr0, _BLK)], idx_v)

            def group(q, carry):
                r0 = q * _RING

                @pl.when(q > 0)
                def _():
                    for j in range(_RING):
                        pltpu.make_async_copy(
                            rows[j], out_hbm.at[pl.ds((fr0 + r0 - _RING + j)
                                                      * _L, _L)], sw[j]).wait()

                for j in range(_RING):
                    pltpu.async_copy(shared_tab.at[idx_v.at[r0 + j]], rows[j],
                                     sg[j])
                for j in range(_RING):
                    fr = fr0 + r0 + j
                    pltpu.make_async_copy(shared_tab.at[idx_v.at[r0 + j]],
                                          rows[j], sg[j]).wait()
                    pltpu.async_copy(rows[j], out_hbm.at[pl.ds(fr * _L, _L)],
                                     sw[j])
                return carry

            lax.fori_loop(0, n_groups, group, 0)
            for j in range(_RING):
                fr = fr0 + _BLK - _RING + j
                pltpu.make_async_copy(rows[j],
                                      out_hbm.at[pl.ds(fr * _L, _L)],
                                      sw[j]).wait()
            return carry

        lax.fori_loop(0, n_blocks, block, 0)

    return gather_kernel


def kernel(x, table):
    bsz, hist = x.shape
    d = table.shape[1]
    b = bsz * hist
    idx = x.reshape(b // _L, _L).astype(jnp.int32)
    out = _build_gather(b, d)(idx, table)
    return out.reshape(bsz, hist, d)
